# baseline (speedup 1.0000x reference)
import functools

import jax
import jax.numpy as jnp
from jax import lax
from jax.experimental import pallas as pl
from jax.experimental.pallas import tpu as pltpu
from jax.experimental.pallas import tpu_sc as plsc

N_USER = 50000
N_MOVIE = 10000
E = 500000
D = 128

NC = 2
NS = 16
NW = NC * NS

B = 128
E_PAD = 524288
EPT = E_PAD // NW
SJ = EPT // B
NB = 4
NG = SJ // NB

NP_U = 51200
NP_M = 10240
NCHUNK = 8
CW = 16

_f32 = jnp.float32


def _mesh():
    return plsc.VectorSubcoreMesh(core_axis_name="c", subcore_axis_name="s")


def _counts_body(dstu_hbm, dstm_hbm, outu_hbm, outm_hbm,
                 du, dm, ones_v, zrow, acc_u, acc_m):
    scid = lax.axis_index("c")
    tid = lax.axis_index("s")
    wid = scid * NS + tid
    e0 = wid * SJ

    pltpu.sync_copy(dstu_hbm.at[pl.ds(e0, SJ)], du)
    pltpu.sync_copy(dstm_hbm.at[pl.ds(e0, SJ)], dm)

    @pl.loop(0, B)
    def _fill(i):
        ones_v[i] = jnp.ones((16,), _f32)
        zrow[i] = jnp.zeros((16,), _f32)

    zu = NP_U // NS
    zm = NP_M // NS

    @pl.loop(0, zu // B)
    def _zu(k):
        pltpu.sync_copy(zrow, acc_u.at[pl.ds(tid * zu + k * B, B)])

    @pl.loop(0, zm // B)
    def _zm(k):
        pltpu.sync_copy(zrow, acc_m.at[pl.ds(tid * zm + k * B, B)])

    plsc.subcore_barrier()

    @pl.loop(0, SJ)
    def _scatter(j):
        pltpu.sync_copy(ones_v, acc_u.at[du.at[j]], add=True)
        pltpu.sync_copy(ones_v, acc_m.at[dm.at[j]], add=True)

    plsc.subcore_barrier()
    pltpu.sync_copy(acc_u.at[pl.ds(tid * zu, zu)],
                    outu_hbm.at[scid, pl.ds(tid * zu, zu)])
    pltpu.sync_copy(acc_m.at[pl.ds(tid * zm, zm)],
                    outm_hbm.at[scid, pl.ds(tid * zm, zm)])


def _make_counts():
    return pl.kernel(
        _counts_body,
        out_type=[jax.ShapeDtypeStruct((NC, NP_U, 16), _f32),
                  jax.ShapeDtypeStruct((NC, NP_M, 16), _f32)],
        mesh=_mesh(),
        scratch_types=[
            pltpu.VMEM((SJ, B), jnp.int32),
            pltpu.VMEM((SJ, B), jnp.int32),
            pltpu.VMEM((B, 16), _f32),
            pltpu.VMEM((B, 16), _f32),
            pltpu.VMEM_SHARED((NP_U, 16), _f32),
            pltpu.VMEM_SHARED((NP_M, 16), _f32),
        ],
        compiler_params=pltpu.CompilerParams(use_tc_tiling_on_sc=False),
    )


def _agg_body(n_src, np_dst, *refs):
    xps = list(refs[:NCHUNK])
    (srcb_hbm, dstb_hbm, out_hbm,
     sidx, didx, rows, zrow, table_s, acc, gs0, gs1, ss0, ss1) = refs[NCHUNK:]
    scid = lax.axis_index("c")
    tid = lax.axis_index("s")
    wid = scid * NS + tid
    e0 = wid * SJ

    pltpu.sync_copy(srcb_hbm.at[pl.ds(e0, SJ)], sidx)
    pltpu.sync_copy(dstb_hbm.at[pl.ds(e0, SJ)], didx)

    @pl.loop(0, B)
    def _fill(i):
        for q in range(CW // 16):
            zrow[i, pl.ds(q * 16, 16)] = jnp.zeros((16,), _f32)

    zpt = np_dst // NS
    tpt = n_src // NS

    for c in range(NCHUNK):
        pltpu.sync_copy(xps[c].at[pl.ds(tid * tpt, tpt)],
                        table_s.at[pl.ds(tid * tpt, tpt)])

        @pl.loop(0, zpt // B)
        def _zero(k):
            pltpu.sync_copy(zrow, acc.at[pl.ds(tid * zpt + k * B, B)])

        plsc.subcore_barrier()

        for b in range(NB):
            pltpu.async_copy(table_s.at[sidx.at[b]], rows.at[0, b], gs0)

        @pl.loop(0, NG // 2)
        def _pair(t):
            g0 = 2 * t
            g1 = 2 * t + 1
            for b in range(NB):
                pltpu.make_async_copy(table_s.at[sidx.at[g0 * NB + b]],
                                      rows.at[0, b], gs0).wait()
            d_g1 = [pltpu.async_copy(table_s.at[sidx.at[g1 * NB + b]],
                                     rows.at[1, b], gs1) for b in range(NB)]
            d_s0 = [pltpu.async_copy(rows.at[0, b],
                                     acc.at[didx.at[g0 * NB + b]],
                                     ss0, add=True) for b in range(NB)]
            for d in d_g1:
                d.wait()
            d_s1 = [pltpu.async_copy(rows.at[1, b],
                                     acc.at[didx.at[g1 * NB + b]],
                                     ss1, add=True) for b in range(NB)]
            for d in d_s0:
                d.wait()

            @pl.when(t < NG // 2 - 1)
            def _next():
                for b in range(NB):
                    pltpu.async_copy(table_s.at[sidx.at[(g1 + 1) * NB + b]],
                                     rows.at[0, b], gs0)

            for d in d_s1:
                d.wait()

        plsc.subcore_barrier()
        pltpu.sync_copy(acc.at[pl.ds(tid * zpt, zpt)],
                        out_hbm.at[c, scid, pl.ds(tid * zpt, zpt)])
        plsc.subcore_barrier()


def _make_agg(n_src, np_dst):
    return pl.kernel(
        functools.partial(_agg_body, n_src, np_dst),
        out_type=jax.ShapeDtypeStruct((NCHUNK, NC, np_dst, CW), _f32),
        mesh=_mesh(),
        scratch_types=[
            pltpu.VMEM((SJ, B), jnp.int32),
            pltpu.VMEM((SJ, B), jnp.int32),
            pltpu.VMEM((2, NB, B, CW), _f32),
            pltpu.VMEM((B, CW), _f32),
            pltpu.VMEM_SHARED((n_src, CW), _f32),
            pltpu.VMEM_SHARED((np_dst, CW), _f32),
            pltpu.SemaphoreType.DMA,
            pltpu.SemaphoreType.DMA,
            pltpu.SemaphoreType.DMA,
            pltpu.SemaphoreType.DMA,
        ],
        compiler_params=pltpu.CompilerParams(use_tc_tiling_on_sc=False),
    )


R = 1000


def _proj_body(x_ref, wl_ref, wr_ref, *outs):
    x = x_ref[...]
    p = jnp.dot(x, wl_ref[...], preferred_element_type=_f32)
    for c in range(NCHUNK):
        outs[c][...] = p[:, c * CW:(c + 1) * CW]
    outs[NCHUNK][...] = jnp.dot(x, wr_ref[...], preferred_element_type=_f32)


def _proj(x, wl, wr):
    n = x.shape[0]
    nb = n // R
    outs = pl.pallas_call(
        _proj_body,
        grid=(nb,),
        in_specs=[
            pl.BlockSpec((R, D), lambda i: (i, 0)),
            pl.BlockSpec((D, D), lambda i: (0, 0)),
            pl.BlockSpec((D, D), lambda i: (0, 0)),
        ],
        out_specs=[pl.BlockSpec((R, CW), lambda i: (i, 0))] * NCHUNK
        + [pl.BlockSpec((R, D), lambda i: (i, 0))],
        out_shape=[jax.ShapeDtypeStruct((n, CW), _f32)] * NCHUNK
        + [jax.ShapeDtypeStruct((n, D), _f32)],
    )(x, wl, wr)
    return outs[:NCHUNK], outs[NCHUNK]


def _merge_agg(p_ref, cnt_ref, xr_ref, b_ref):
    agg = jnp.concatenate(
        [p_ref[c, 0] + p_ref[c, 1] for c in range(NCHUNK)], axis=-1)
    cnt = cnt_ref[0, :, 0:1] + cnt_ref[1, :, 0:1]
    rc = 1.0 / jnp.maximum(cnt, 1.0)
    return agg * rc + b_ref[...] + xr_ref[...]


def _combine_proj_body(p_ref, cnt_ref, xr_ref, b_ref, wl_ref, wr_ref, *outs):
    o = _merge_agg(p_ref, cnt_ref, xr_ref, b_ref)
    h = jnp.where(o >= 0.0, o, 0.01 * o)
    p = jnp.dot(h, wl_ref[...], preferred_element_type=_f32)
    for c in range(NCHUNK):
        outs[c][...] = p[:, c * CW:(c + 1) * CW]
    outs[NCHUNK][...] = jnp.dot(h, wr_ref[...], preferred_element_type=_f32)


def _combine_proj(parts, cnt, xr, bvec, wl, wr, n, np_dst):
    nb = n // R
    outs = pl.pallas_call(
        _combine_proj_body,
        grid=(nb,),
        in_specs=[
            pl.BlockSpec((NCHUNK, NC, R, CW), lambda i: (0, 0, i, 0)),
            pl.BlockSpec((NC, R, 16), lambda i: (0, i, 0)),
            pl.BlockSpec((R, D), lambda i: (i, 0)),
            pl.BlockSpec((1, D), lambda i: (0, 0)),
            pl.BlockSpec((D, D), lambda i: (0, 0)),
            pl.BlockSpec((D, D), lambda i: (0, 0)),
        ],
        out_specs=[pl.BlockSpec((R, CW), lambda i: (i, 0))] * NCHUNK
        + [pl.BlockSpec((R, D), lambda i: (i, 0))],
        out_shape=[jax.ShapeDtypeStruct((n, CW), _f32)] * NCHUNK
        + [jax.ShapeDtypeStruct((n, D), _f32)],
    )(parts, cnt, xr, bvec, wl, wr)
    return outs[:NCHUNK], outs[NCHUNK]


def _final_body(p_ref, cnt_ref, xr_ref, b_ref, out_ref):
    out_ref[...] = _merge_agg(p_ref, cnt_ref, xr_ref, b_ref)


def _final(parts, cnt, xr, bvec, n):
    nb = n // R
    return pl.pallas_call(
        _final_body,
        grid=(nb,),
        in_specs=[
            pl.BlockSpec((NCHUNK, NC, R, CW), lambda i: (0, 0, i, 0)),
            pl.BlockSpec((NC, R, 16), lambda i: (0, i, 0)),
            pl.BlockSpec((R, D), lambda i: (i, 0)),
            pl.BlockSpec((1, D), lambda i: (0, 0)),
        ],
        out_specs=pl.BlockSpec((R, D), lambda i: (i, 0)),
        out_shape=jax.ShapeDtypeStruct((n, D), _f32),
    )(parts, cnt, xr, bvec)


def kernel(x_user, x_movie, edge_index,
           Wl_um_0, b_um_0, Wr_um_0, Wl_mu_0, b_mu_0, Wr_mu_0,
           Wl_um_1, b_um_1, Wr_um_1, Wl_mu_1, b_mu_1, Wr_mu_1):
    u_idx = edge_index[0].astype(jnp.int32)
    m_idx = edge_index[1].astype(jnp.int32)
    npad = E_PAD - E
    pad0 = jnp.zeros((npad,), jnp.int32)
    spread = jnp.arange(npad, dtype=jnp.int32)
    dpad_m = N_MOVIE + (spread % (NP_M - N_MOVIE - 8))
    dpad_u = N_USER + (spread % (NP_U - N_USER - 8))
    srcb_m = jnp.concatenate([u_idx, pad0]).reshape(E_PAD // B, B)
    dstb_m = jnp.concatenate([m_idx, dpad_m]).reshape(E_PAD // B, B)
    srcb_u = jnp.concatenate([m_idx, pad0]).reshape(E_PAD // B, B)
    dstb_u = jnp.concatenate([u_idx, dpad_u]).reshape(E_PAD // B, B)

    cnt_u, cnt_m = _make_counts()(dstb_u, dstb_m)

    agg_to_movie = _make_agg(N_USER, NP_M)
    agg_to_user = _make_agg(N_MOVIE, NP_U)

    xpu0, xr_u0 = _proj(x_user, Wl_um_0, Wr_mu_0)
    xpm0, xr_m0 = _proj(x_movie, Wl_mu_0, Wr_um_0)

    pm0 = agg_to_movie(*xpu0, srcb_m, dstb_m)
    pu0 = agg_to_user(*xpm0, srcb_u, dstb_u)

    xpu1, xr_u1 = _combine_proj(pu0, cnt_u, xr_u0, b_mu_0.reshape(1, D),
                                Wl_um_1, Wr_mu_1, N_USER, NP_U)
    xpm1, xr_m1 = _combine_proj(pm0, cnt_m, xr_m0, b_um_0.reshape(1, D),
                                Wl_mu_1, Wr_um_1, N_MOVIE, NP_M)

    pu1 = agg_to_user(*xpm1, srcb_u, dstb_u)
    pm1 = agg_to_movie(*xpu1, srcb_m, dstb_m)

    out_user = _final(pu1, cnt_u, xr_u1, b_mu_1.reshape(1, D), N_USER)
    out_movie = _final(pm1, cnt_m, xr_m1, b_um_1.reshape(1, D), N_MOVIE)
    return (out_user, out_movie)


# speedup vs baseline: 1.1866x; 1.1866x over previous
import functools

import jax
import jax.numpy as jnp
from jax import lax
from jax.experimental import pallas as pl
from jax.experimental.pallas import tpu as pltpu
from jax.experimental.pallas import tpu_sc as plsc

N_USER = 50000
N_MOVIE = 10000
E = 500000
D = 128

NC = 2
NS = 16
NW = NC * NS

B = 128
E_PAD = 524288
EPT = E_PAD // NW
SJ = EPT // B
NB = 4
NG = SJ // NB

NP_U = 51200
NP_M = 10240
NCHUNK = 8
CW = 16

_f32 = jnp.float32


def _mesh():
    return plsc.VectorSubcoreMesh(core_axis_name="c", subcore_axis_name="s")


def _counts_body(dstu_hbm, dstm_hbm, outu_hbm, outm_hbm,
                 du, dm, ones_v, zrow, acc_u, acc_m):
    scid = lax.axis_index("c")
    tid = lax.axis_index("s")
    wid = scid * NS + tid
    e0 = wid * SJ

    pltpu.sync_copy(dstu_hbm.at[pl.ds(e0, SJ)], du)
    pltpu.sync_copy(dstm_hbm.at[pl.ds(e0, SJ)], dm)

    @pl.loop(0, B)
    def _fill(i):
        ones_v[i] = jnp.ones((16,), _f32)
        zrow[i] = jnp.zeros((16,), _f32)

    zu = NP_U // NS
    zm = NP_M // NS

    @pl.loop(0, zu // B)
    def _zu(k):
        pltpu.sync_copy(zrow, acc_u.at[pl.ds(tid * zu + k * B, B)])

    @pl.loop(0, zm // B)
    def _zm(k):
        pltpu.sync_copy(zrow, acc_m.at[pl.ds(tid * zm + k * B, B)])

    plsc.subcore_barrier()

    @pl.loop(0, SJ)
    def _scatter(j):
        pltpu.sync_copy(ones_v, acc_u.at[du.at[j]], add=True)
        pltpu.sync_copy(ones_v, acc_m.at[dm.at[j]], add=True)

    plsc.subcore_barrier()
    pltpu.sync_copy(acc_u.at[pl.ds(tid * zu, zu)],
                    outu_hbm.at[scid, pl.ds(tid * zu, zu)])
    pltpu.sync_copy(acc_m.at[pl.ds(tid * zm, zm)],
                    outm_hbm.at[scid, pl.ds(tid * zm, zm)])


def _make_counts():
    return pl.kernel(
        _counts_body,
        out_type=[jax.ShapeDtypeStruct((NC, NP_U, 16), _f32),
                  jax.ShapeDtypeStruct((NC, NP_M, 16), _f32)],
        mesh=_mesh(),
        scratch_types=[
            pltpu.VMEM((SJ, B), jnp.int32),
            pltpu.VMEM((SJ, B), jnp.int32),
            pltpu.VMEM((B, 16), _f32),
            pltpu.VMEM((B, 16), _f32),
            pltpu.VMEM_SHARED((NP_U, 16), _f32),
            pltpu.VMEM_SHARED((NP_M, 16), _f32),
        ],
        compiler_params=pltpu.CompilerParams(use_tc_tiling_on_sc=False),
    )


def _agg_body(n_src, np_dst, *refs):
    (xp_hbm, srcb_hbm, dstb_hbm, out_hbm,
     sidx, didx, rows, zrow, table_s, acc, gs0, gs1, ss0, ss1) = refs
    scid = lax.axis_index("c")
    tid = lax.axis_index("s")
    wid = scid * NS + tid
    e0 = wid * SJ

    pltpu.sync_copy(srcb_hbm.at[pl.ds(e0, SJ)], sidx)
    pltpu.sync_copy(dstb_hbm.at[pl.ds(e0, SJ)], didx)

    @pl.loop(0, B)
    def _fill(i):
        for q in range(CW // 16):
            zrow[i, pl.ds(q * 16, 16)] = jnp.zeros((16,), _f32)

    zpt = np_dst // NS
    tpt = -(-(n_src // NS) // 8) * 8
    tpt_last = n_src - 15 * tpt

    for c in range(NCHUNK):
        @pl.when(tid < NS - 1)
        def _ld():
            pltpu.sync_copy(
                xp_hbm.at[pl.ds(tid * tpt, tpt), pl.ds(c * CW, CW)],
                table_s.at[pl.ds(tid * tpt, tpt)])

        @pl.when(tid == NS - 1)
        def _ld_last():
            pltpu.sync_copy(
                xp_hbm.at[pl.ds((NS - 1) * tpt, tpt_last), pl.ds(c * CW, CW)],
                table_s.at[pl.ds((NS - 1) * tpt, tpt_last)])

        @pl.loop(0, zpt // B)
        def _zero(k):
            pltpu.sync_copy(zrow, acc.at[pl.ds(tid * zpt + k * B, B)])

        plsc.subcore_barrier()

        for b in range(NB):
            pltpu.async_copy(table_s.at[sidx.at[b]], rows.at[0, b], gs0)

        @pl.loop(0, NG // 2)
        def _pair(t):
            g0 = 2 * t
            g1 = 2 * t + 1
            for b in range(NB):
                pltpu.make_async_copy(table_s.at[sidx.at[g0 * NB + b]],
                                      rows.at[0, b], gs0).wait()
            d_g1 = [pltpu.async_copy(table_s.at[sidx.at[g1 * NB + b]],
                                     rows.at[1, b], gs1) for b in range(NB)]
            d_s0 = [pltpu.async_copy(rows.at[0, b],
                                     acc.at[didx.at[g0 * NB + b]],
                                     ss0, add=True) for b in range(NB)]
            for d in d_g1:
                d.wait()
            d_s1 = [pltpu.async_copy(rows.at[1, b],
                                     acc.at[didx.at[g1 * NB + b]],
                                     ss1, add=True) for b in range(NB)]
            for d in d_s0:
                d.wait()

            @pl.when(t < NG // 2 - 1)
            def _next():
                for b in range(NB):
                    pltpu.async_copy(table_s.at[sidx.at[(g1 + 1) * NB + b]],
                                     rows.at[0, b], gs0)

            for d in d_s1:
                d.wait()

        plsc.subcore_barrier()
        pltpu.sync_copy(acc.at[pl.ds(tid * zpt, zpt)],
                        out_hbm.at[scid, pl.ds(tid * zpt, zpt),
                                   pl.ds(c * CW, CW)])
        plsc.subcore_barrier()


def _make_agg(n_src, np_dst):
    return pl.kernel(
        functools.partial(_agg_body, n_src, np_dst),
        out_type=jax.ShapeDtypeStruct((NC, np_dst, D), _f32),
        mesh=_mesh(),
        scratch_types=[
            pltpu.VMEM((SJ, B), jnp.int32),
            pltpu.VMEM((SJ, B), jnp.int32),
            pltpu.VMEM((2, NB, B, CW), _f32),
            pltpu.VMEM((B, CW), _f32),
            pltpu.VMEM_SHARED((n_src, CW), _f32),
            pltpu.VMEM_SHARED((np_dst, CW), _f32),
            pltpu.SemaphoreType.DMA,
            pltpu.SemaphoreType.DMA,
            pltpu.SemaphoreType.DMA,
            pltpu.SemaphoreType.DMA,
        ],
        compiler_params=pltpu.CompilerParams(use_tc_tiling_on_sc=False),
    )


R = 1000


def _proj_body(x_ref, wl_ref, wr_ref, xp_ref, xr_ref):
    x = x_ref[...]
    xp_ref[...] = jnp.dot(x, wl_ref[...], preferred_element_type=_f32)
    xr_ref[...] = jnp.dot(x, wr_ref[...], preferred_element_type=_f32)


def _proj(x, wl, wr):
    n = x.shape[0]
    nb = n // R
    return pl.pallas_call(
        _proj_body,
        grid=(nb,),
        in_specs=[
            pl.BlockSpec((R, D), lambda i: (i, 0)),
            pl.BlockSpec((D, D), lambda i: (0, 0)),
            pl.BlockSpec((D, D), lambda i: (0, 0)),
        ],
        out_specs=[pl.BlockSpec((R, D), lambda i: (i, 0))] * 2,
        out_shape=[jax.ShapeDtypeStruct((n, D), _f32)] * 2,
    )(x, wl, wr)


def _merge_agg(p_ref, cnt_ref, xr_ref, b_ref):
    agg = p_ref[0] + p_ref[1]
    cnt = cnt_ref[0, :, 0:1] + cnt_ref[1, :, 0:1]
    rc = 1.0 / jnp.maximum(cnt, 1.0)
    return agg * rc + b_ref[...] + xr_ref[...]


def _combine_proj_body(p_ref, cnt_ref, xr_ref, b_ref, wl_ref, wr_ref,
                       xp_ref, xr_out):
    o = _merge_agg(p_ref, cnt_ref, xr_ref, b_ref)
    h = jnp.where(o >= 0.0, o, 0.01 * o)
    xp_ref[...] = jnp.dot(h, wl_ref[...], preferred_element_type=_f32)
    xr_out[...] = jnp.dot(h, wr_ref[...], preferred_element_type=_f32)


def _combine_proj(parts, cnt, xr, bvec, wl, wr, n):
    nb = n // R
    return pl.pallas_call(
        _combine_proj_body,
        grid=(nb,),
        in_specs=[
            pl.BlockSpec((NC, R, D), lambda i: (0, i, 0)),
            pl.BlockSpec((NC, R, 16), lambda i: (0, i, 0)),
            pl.BlockSpec((R, D), lambda i: (i, 0)),
            pl.BlockSpec((1, D), lambda i: (0, 0)),
            pl.BlockSpec((D, D), lambda i: (0, 0)),
            pl.BlockSpec((D, D), lambda i: (0, 0)),
        ],
        out_specs=[pl.BlockSpec((R, D), lambda i: (i, 0))] * 2,
        out_shape=[jax.ShapeDtypeStruct((n, D), _f32)] * 2,
    )(parts, cnt, xr, bvec, wl, wr)


def _final_body(p_ref, cnt_ref, xr_ref, b_ref, out_ref):
    out_ref[...] = _merge_agg(p_ref, cnt_ref, xr_ref, b_ref)


def _final(parts, cnt, xr, bvec, n):
    nb = n // R
    return pl.pallas_call(
        _final_body,
        grid=(nb,),
        in_specs=[
            pl.BlockSpec((NC, R, D), lambda i: (0, i, 0)),
            pl.BlockSpec((NC, R, 16), lambda i: (0, i, 0)),
            pl.BlockSpec((R, D), lambda i: (i, 0)),
            pl.BlockSpec((1, D), lambda i: (0, 0)),
        ],
        out_specs=pl.BlockSpec((R, D), lambda i: (i, 0)),
        out_shape=jax.ShapeDtypeStruct((n, D), _f32),
    )(parts, cnt, xr, bvec)


def kernel(x_user, x_movie, edge_index,
           Wl_um_0, b_um_0, Wr_um_0, Wl_mu_0, b_mu_0, Wr_mu_0,
           Wl_um_1, b_um_1, Wr_um_1, Wl_mu_1, b_mu_1, Wr_mu_1):
    u_idx = edge_index[0].astype(jnp.int32)
    m_idx = edge_index[1].astype(jnp.int32)
    npad = E_PAD - E
    pad0 = jnp.zeros((npad,), jnp.int32)
    spread = jnp.arange(npad, dtype=jnp.int32)
    dpad_m = N_MOVIE + (spread % (NP_M - N_MOVIE - 8))
    dpad_u = N_USER + (spread % (NP_U - N_USER - 8))
    srcb_m = jnp.concatenate([u_idx, pad0]).reshape(E_PAD // B, B)
    dstb_m = jnp.concatenate([m_idx, dpad_m]).reshape(E_PAD // B, B)
    srcb_u = jnp.concatenate([m_idx, pad0]).reshape(E_PAD // B, B)
    dstb_u = jnp.concatenate([u_idx, dpad_u]).reshape(E_PAD // B, B)

    cnt_u, cnt_m = _make_counts()(dstb_u, dstb_m)

    agg_to_movie = _make_agg(N_USER, NP_M)
    agg_to_user = _make_agg(N_MOVIE, NP_U)

    xpu0, xr_u0 = _proj(x_user, Wl_um_0, Wr_mu_0)
    xpm0, xr_m0 = _proj(x_movie, Wl_mu_0, Wr_um_0)

    pm0 = agg_to_movie(xpu0, srcb_m, dstb_m)
    pu0 = agg_to_user(xpm0, srcb_u, dstb_u)

    xpu1, xr_u1 = _combine_proj(pu0, cnt_u, xr_u0, b_mu_0.reshape(1, D),
                                Wl_um_1, Wr_mu_1, N_USER)
    xpm1, xr_m1 = _combine_proj(pm0, cnt_m, xr_m0, b_um_0.reshape(1, D),
                                Wl_mu_1, Wr_um_1, N_MOVIE)

    pu1 = agg_to_user(xpm1, srcb_u, dstb_u)
    pm1 = agg_to_movie(xpu1, srcb_m, dstb_m)

    out_user = _final(pu1, cnt_u, xr_u1, b_mu_1.reshape(1, D), N_USER)
    out_movie = _final(pm1, cnt_m, xr_m1, b_um_1.reshape(1, D), N_MOVIE)
    return (out_user, out_movie)


# speedup vs baseline: 1.2671x; 1.0679x over previous
import functools

import jax
import jax.numpy as jnp
from jax import lax
from jax.experimental import pallas as pl
from jax.experimental.pallas import tpu as pltpu
from jax.experimental.pallas import tpu_sc as plsc

N_USER = 50000
N_MOVIE = 10000
E = 500000
D = 128

NC = 2
NS = 16
NW = NC * NS

B = 128
E_PAD = 524288
NB = 4
SJ0 = 144
SJ1 = 112
SJ = 128

NP_U = 51200
NP_M = 10240
NCHUNK = 8
CW = 16

_f32 = jnp.float32


def _mesh():
    return plsc.VectorSubcoreMesh(core_axis_name="c", subcore_axis_name="s")


def _counts_body(dstu_hbm, dstm_hbm, outu_hbm, outm_hbm,
                 du, dm, ones_v, zrow, acc_u, acc_m):
    scid = lax.axis_index("c")
    tid = lax.axis_index("s")
    wid = scid * NS + tid
    e0 = wid * SJ

    pltpu.sync_copy(dstu_hbm.at[pl.ds(e0, SJ)], du)
    pltpu.sync_copy(dstm_hbm.at[pl.ds(e0, SJ)], dm)

    @pl.loop(0, B)
    def _fill(i):
        ones_v[i] = jnp.ones((16,), _f32)
        zrow[i] = jnp.zeros((16,), _f32)

    zu = NP_U // NS
    zm = NP_M // NS

    @pl.loop(0, zu // B)
    def _zu(k):
        pltpu.sync_copy(zrow, acc_u.at[pl.ds(tid * zu + k * B, B)])

    @pl.loop(0, zm // B)
    def _zm(k):
        pltpu.sync_copy(zrow, acc_m.at[pl.ds(tid * zm + k * B, B)])

    plsc.subcore_barrier()

    @pl.loop(0, SJ)
    def _scatter(j):
        pltpu.sync_copy(ones_v, acc_u.at[du.at[j]], add=True)
        pltpu.sync_copy(ones_v, acc_m.at[dm.at[j]], add=True)

    plsc.subcore_barrier()
    pltpu.sync_copy(acc_u.at[pl.ds(tid * zu, zu)],
                    outu_hbm.at[scid, pl.ds(tid * zu, zu)])
    pltpu.sync_copy(acc_m.at[pl.ds(tid * zm, zm)],
                    outm_hbm.at[scid, pl.ds(tid * zm, zm)])


def _make_counts():
    return pl.kernel(
        _counts_body,
        out_type=[jax.ShapeDtypeStruct((NC, NP_U, 16), _f32),
                  jax.ShapeDtypeStruct((NC, NP_M, 16), _f32)],
        mesh=_mesh(),
        scratch_types=[
            pltpu.VMEM((SJ, B), jnp.int32),
            pltpu.VMEM((SJ, B), jnp.int32),
            pltpu.VMEM((B, 16), _f32),
            pltpu.VMEM((B, 16), _f32),
            pltpu.VMEM_SHARED((NP_U, 16), _f32),
            pltpu.VMEM_SHARED((NP_M, 16), _f32),
        ],
        compiler_params=pltpu.CompilerParams(use_tc_tiling_on_sc=False),
    )


def _agg_body(n_src, np_dst, *refs):
    (xp_hbm, srcb_hbm, dstb_hbm, out_hbm,
     sidx, didx, rows, zrow, table_s, acc, gs0, gs1, ss0, ss1) = refs
    scid = lax.axis_index("c")
    tid = lax.axis_index("s")
    e0 = jnp.where(scid == 0, tid * SJ0, NS * SJ0 + tid * SJ1)
    ng2 = jnp.where(scid == 0, SJ0 // (2 * NB), SJ1 // (2 * NB))

    @pl.when(scid == 0)
    def _ld_idx0():
        pltpu.sync_copy(srcb_hbm.at[pl.ds(e0, SJ0)], sidx.at[pl.ds(0, SJ0)])
        pltpu.sync_copy(dstb_hbm.at[pl.ds(e0, SJ0)], didx.at[pl.ds(0, SJ0)])

    @pl.when(scid == 1)
    def _ld_idx1():
        pltpu.sync_copy(srcb_hbm.at[pl.ds(e0, SJ1)], sidx.at[pl.ds(0, SJ1)])
        pltpu.sync_copy(dstb_hbm.at[pl.ds(e0, SJ1)], didx.at[pl.ds(0, SJ1)])

    @pl.loop(0, B)
    def _fill(i):
        for q in range(CW // 16):
            zrow[i, pl.ds(q * 16, 16)] = jnp.zeros((16,), _f32)

    zpt = np_dst // NS
    tpt = -(-(n_src // NS) // 8) * 8
    tpt_last = n_src - 15 * tpt

    for c in range(NCHUNK):
        @pl.when(tid < NS - 1)
        def _ld():
            pltpu.sync_copy(
                xp_hbm.at[pl.ds(tid * tpt, tpt), pl.ds(c * CW, CW)],
                table_s.at[pl.ds(tid * tpt, tpt)])

        @pl.when(tid == NS - 1)
        def _ld_last():
            pltpu.sync_copy(
                xp_hbm.at[pl.ds((NS - 1) * tpt, tpt_last), pl.ds(c * CW, CW)],
                table_s.at[pl.ds((NS - 1) * tpt, tpt_last)])

        @pl.loop(0, zpt // B)
        def _zero(k):
            pltpu.sync_copy(zrow, acc.at[pl.ds(tid * zpt + k * B, B)])

        plsc.subcore_barrier()

        for b in range(NB):
            pltpu.async_copy(table_s.at[sidx.at[b]], rows.at[0, b], gs0)

        @pl.loop(0, ng2)
        def _pair(t):
            g0 = 2 * t
            g1 = 2 * t + 1
            for b in range(NB):
                pltpu.make_async_copy(table_s.at[sidx.at[g0 * NB + b]],
                                      rows.at[0, b], gs0).wait()
            d_g1 = [pltpu.async_copy(table_s.at[sidx.at[g1 * NB + b]],
                                     rows.at[1, b], gs1) for b in range(NB)]
            d_s0 = [pltpu.async_copy(rows.at[0, b],
                                     acc.at[didx.at[g0 * NB + b]],
                                     ss0, add=True) for b in range(NB)]
            for d in d_g1:
                d.wait()
            d_s1 = [pltpu.async_copy(rows.at[1, b],
                                     acc.at[didx.at[g1 * NB + b]],
                                     ss1, add=True) for b in range(NB)]
            for d in d_s0:
                d.wait()

            @pl.when(t < ng2 - 1)
            def _next():
                for b in range(NB):
                    pltpu.async_copy(table_s.at[sidx.at[(g1 + 1) * NB + b]],
                                     rows.at[0, b], gs0)

            for d in d_s1:
                d.wait()

        plsc.subcore_barrier()
        pltpu.sync_copy(acc.at[pl.ds(tid * zpt, zpt)],
                        out_hbm.at[scid, pl.ds(tid * zpt, zpt),
                                   pl.ds(c * CW, CW)])
        plsc.subcore_barrier()


def _make_agg(n_src, np_dst):
    return pl.kernel(
        functools.partial(_agg_body, n_src, np_dst),
        out_type=jax.ShapeDtypeStruct((NC, np_dst, D), _f32),
        mesh=_mesh(),
        scratch_types=[
            pltpu.VMEM((SJ0, B), jnp.int32),
            pltpu.VMEM((SJ0, B), jnp.int32),
            pltpu.VMEM((2, NB, B, CW), _f32),
            pltpu.VMEM((B, CW), _f32),
            pltpu.VMEM_SHARED((n_src, CW), _f32),
            pltpu.VMEM_SHARED((np_dst, CW), _f32),
            pltpu.SemaphoreType.DMA,
            pltpu.SemaphoreType.DMA,
            pltpu.SemaphoreType.DMA,
            pltpu.SemaphoreType.DMA,
        ],
        compiler_params=pltpu.CompilerParams(use_tc_tiling_on_sc=False),
    )


R = 1000


def _proj_body(x_ref, wl_ref, wr_ref, xp_ref, xr_ref):
    x = x_ref[...]
    xp_ref[...] = jnp.dot(x, wl_ref[...], preferred_element_type=_f32)
    xr_ref[...] = jnp.dot(x, wr_ref[...], preferred_element_type=_f32)


def _proj(x, wl, wr):
    n = x.shape[0]
    nb = n // R
    return pl.pallas_call(
        _proj_body,
        grid=(nb,),
        in_specs=[
            pl.BlockSpec((R, D), lambda i: (i, 0)),
            pl.BlockSpec((D, D), lambda i: (0, 0)),
            pl.BlockSpec((D, D), lambda i: (0, 0)),
        ],
        out_specs=[pl.BlockSpec((R, D), lambda i: (i, 0))] * 2,
        out_shape=[jax.ShapeDtypeStruct((n, D), _f32)] * 2,
    )(x, wl, wr)


def _merge_agg(p_ref, cnt_ref, xr_ref, b_ref):
    agg = p_ref[0] + p_ref[1]
    cnt = cnt_ref[0, :, 0:1] + cnt_ref[1, :, 0:1]
    rc = 1.0 / jnp.maximum(cnt, 1.0)
    return agg * rc + b_ref[...] + xr_ref[...]


def _combine_proj_body(p_ref, cnt_ref, xr_ref, b_ref, wl_ref, wr_ref,
                       xp_ref, xr_out):
    o = _merge_agg(p_ref, cnt_ref, xr_ref, b_ref)
    h = jnp.where(o >= 0.0, o, 0.01 * o)
    xp_ref[...] = jnp.dot(h, wl_ref[...], preferred_element_type=_f32)
    xr_out[...] = jnp.dot(h, wr_ref[...], preferred_element_type=_f32)


def _combine_proj(parts, cnt, xr, bvec, wl, wr, n):
    nb = n // R
    return pl.pallas_call(
        _combine_proj_body,
        grid=(nb,),
        in_specs=[
            pl.BlockSpec((NC, R, D), lambda i: (0, i, 0)),
            pl.BlockSpec((NC, R, 16), lambda i: (0, i, 0)),
            pl.BlockSpec((R, D), lambda i: (i, 0)),
            pl.BlockSpec((1, D), lambda i: (0, 0)),
            pl.BlockSpec((D, D), lambda i: (0, 0)),
            pl.BlockSpec((D, D), lambda i: (0, 0)),
        ],
        out_specs=[pl.BlockSpec((R, D), lambda i: (i, 0))] * 2,
        out_shape=[jax.ShapeDtypeStruct((n, D), _f32)] * 2,
    )(parts, cnt, xr, bvec, wl, wr)


def _final_body(p_ref, cnt_ref, xr_ref, b_ref, out_ref):
    out_ref[...] = _merge_agg(p_ref, cnt_ref, xr_ref, b_ref)


def _final(parts, cnt, xr, bvec, n):
    nb = n // R
    return pl.pallas_call(
        _final_body,
        grid=(nb,),
        in_specs=[
            pl.BlockSpec((NC, R, D), lambda i: (0, i, 0)),
            pl.BlockSpec((NC, R, 16), lambda i: (0, i, 0)),
            pl.BlockSpec((R, D), lambda i: (i, 0)),
            pl.BlockSpec((1, D), lambda i: (0, 0)),
        ],
        out_specs=pl.BlockSpec((R, D), lambda i: (i, 0)),
        out_shape=jax.ShapeDtypeStruct((n, D), _f32),
    )(parts, cnt, xr, bvec)


def kernel(x_user, x_movie, edge_index,
           Wl_um_0, b_um_0, Wr_um_0, Wl_mu_0, b_mu_0, Wr_mu_0,
           Wl_um_1, b_um_1, Wr_um_1, Wl_mu_1, b_mu_1, Wr_mu_1):
    u_idx = edge_index[0].astype(jnp.int32)
    m_idx = edge_index[1].astype(jnp.int32)
    npad = E_PAD - E
    pad0 = jnp.zeros((npad,), jnp.int32)
    spread = jnp.arange(npad, dtype=jnp.int32)
    dpad_m = N_MOVIE + (spread % (NP_M - N_MOVIE - 8))
    dpad_u = N_USER + (spread % (NP_U - N_USER - 8))
    srcb_m = jnp.concatenate([u_idx, pad0]).reshape(E_PAD // B, B)
    dstb_m = jnp.concatenate([m_idx, dpad_m]).reshape(E_PAD // B, B)
    srcb_u = jnp.concatenate([m_idx, pad0]).reshape(E_PAD // B, B)
    dstb_u = jnp.concatenate([u_idx, dpad_u]).reshape(E_PAD // B, B)

    cnt_u, cnt_m = _make_counts()(dstb_u, dstb_m)

    agg_to_movie = _make_agg(N_USER, NP_M)
    agg_to_user = _make_agg(N_MOVIE, NP_U)

    xpu0, xr_u0 = _proj(x_user, Wl_um_0, Wr_mu_0)
    xpm0, xr_m0 = _proj(x_movie, Wl_mu_0, Wr_um_0)

    pm0 = agg_to_movie(xpu0, srcb_m, dstb_m)
    pu0 = agg_to_user(xpm0, srcb_u, dstb_u)

    xpu1, xr_u1 = _combine_proj(pu0, cnt_u, xr_u0, b_mu_0.reshape(1, D),
                                Wl_um_1, Wr_mu_1, N_USER)
    xpm1, xr_m1 = _combine_proj(pm0, cnt_m, xr_m0, b_um_0.reshape(1, D),
                                Wl_mu_1, Wr_um_1, N_MOVIE)

    pu1 = agg_to_user(xpm1, srcb_u, dstb_u)
    pm1 = agg_to_movie(xpu1, srcb_m, dstb_m)

    out_user = _final(pu1, cnt_u, xr_u1, b_mu_1.reshape(1, D), N_USER)
    out_movie = _final(pm1, cnt_m, xr_m1, b_um_1.reshape(1, D), N_MOVIE)
    return (out_user, out_movie)
